# baseline (speedup 1.0000x reference)
import functools

import jax
import jax.numpy as jnp
from jax import lax
from jax.experimental import pallas as pl
from jax.experimental.pallas import tpu as pltpu
from jax.experimental.pallas import tpu_sc as plsc

L = 3
NF = 128
ATTR = 4
STATE = 3
BS = 2
N = 1000
E = 8000
OUT = 3

NP = 1024
NC, NS = 2, 16
NW = NC * NS
DUMP = 1000
NPAD = 1024
CE = 125

_EB = 4000


def _extract_body(rr_ref, rs_ref, idx_ref):
    b = pl.program_id(0)
    base = b * NP
    io = lax.broadcasted_iota(jnp.int32, (1, NPAD), 1)
    r = jnp.max(rr_ref[0].astype(jnp.int32) * io, axis=-1)
    s = jnp.max(rs_ref[0].astype(jnp.int32) * io, axis=-1)
    fill = jnp.broadcast_to(base, (8, 4, 128 - CE))
    idx_ref[:, 0:4, 0:CE] = r.reshape(8, 4, CE) + base
    idx_ref[:, 0:4, CE:128] = fill
    idx_ref[:, 4:8, 0:CE] = s.reshape(8, 4, CE) + base
    idx_ref[:, 4:8, CE:128] = fill
    idx_ref[:, 8:12, 0:CE] = r.reshape(8, 4, CE)
    idx_ref[:, 8:12, CE:128] = jnp.full((8, 4, 128 - CE), DUMP, jnp.int32)


def _extract_indices(Rr8, Rs8):
    return pl.pallas_call(
        _extract_body,
        grid=(BS, E // _EB),
        in_specs=[
            pl.BlockSpec((1, _EB, NPAD), lambda b, i: (b, i, 0)),
            pl.BlockSpec((1, _EB, NPAD), lambda b, i: (b, i, 0)),
        ],
        out_specs=pl.BlockSpec((8, 12, 128), lambda b, i: (b * 2 + i, 0, 0)),
        out_shape=jax.ShapeDtypeStruct((NW, 12, 128), jnp.int32),
        compiler_params=pltpu.CompilerParams(
            dimension_semantics=("parallel", "parallel")),
    )(Rr8, Rs8)


def _dot(a, b):
    return jnp.dot(a, b, preferred_element_type=jnp.float32)


def _first_body(a_ref, st_ref, wn1, wn2, bn, we1, we2, wu1,
                hw1_ref, hw2_ref, hu1_ref):
    h = jnp.maximum(_dot(a_ref[...], wn1[...])
                    + _dot(st_ref[...], wn2[...]) + bn[...], 0.0)
    hw1_ref[...] = _dot(h, we1[...])
    hw2_ref[...] = _dot(h, we2[...])
    hu1_ref[...] = _dot(h, wu1[...])


def _tc_first(attr2, state2, wn, bn, we, wu):
    o = jax.ShapeDtypeStruct((BS * NP, NF), jnp.float32)
    return pl.pallas_call(
        _first_body,
        grid=(2,),
        in_specs=[
            pl.BlockSpec((NP, ATTR), lambda i: (i, 0)),
            pl.BlockSpec((NP, STATE), lambda i: (i, 0)),
            pl.BlockSpec((ATTR, NF), lambda i: (0, 0)),
            pl.BlockSpec((STATE, NF), lambda i: (0, 0)),
            pl.BlockSpec((1, NF), lambda i: (0, 0)),
            pl.BlockSpec((NF, NF), lambda i: (0, 0)),
            pl.BlockSpec((NF, NF), lambda i: (0, 0)),
            pl.BlockSpec((NF, NF), lambda i: (0, 0)),
        ],
        out_specs=[pl.BlockSpec((NP, NF), lambda i: (i, 0))] * 3,
        out_shape=[o, o, o],
        compiler_params=pltpu.CompilerParams(
            dimension_semantics=("parallel",)),
    )(attr2, state2, wn[:ATTR], wn[ATTR:ATTR + STATE], bn.reshape(1, NF),
      we[:NF], we[NF:], wu[:NF])


def _mid_body(pe_ref, hu1_ref, agg_ref, wu2, bu, a_ref, st_ref,
              wn1, wn2, wn3, bn, we1, we2, wu1,
              pe_o, hw1_o, hw2_o, hu1_o):
    upd = jnp.maximum(hu1_ref[...] + _dot(agg_ref[...], wu2[...]) + bu[...],
                      0.0)
    pe = pe_ref[...] + upd
    pe_o[...] = pe
    h = jnp.maximum(_dot(a_ref[...], wn1[...])
                    + _dot(st_ref[...], wn2[...])
                    + _dot(pe, wn3[...]) + bn[...], 0.0)
    hw1_o[...] = _dot(h, we1[...])
    hw2_o[...] = _dot(h, we2[...])
    hu1_o[...] = _dot(h, wu1[...])


def _tc_mid(pe, hu1, agg, wu2, bu, attr2, state2, wn, bn, we, wu):
    o = jax.ShapeDtypeStruct((BS * NP, NF), jnp.float32)
    full = pl.BlockSpec((NF, NF), lambda i: (0, 0))
    row = pl.BlockSpec((NP, NF), lambda i: (i, 0))
    bias = pl.BlockSpec((1, NF), lambda i: (0, 0))
    return pl.pallas_call(
        _mid_body,
        grid=(2,),
        in_specs=[
            row, row, row, full, bias,
            pl.BlockSpec((NP, ATTR), lambda i: (i, 0)),
            pl.BlockSpec((NP, STATE), lambda i: (i, 0)),
            pl.BlockSpec((ATTR, NF), lambda i: (0, 0)),
            pl.BlockSpec((STATE, NF), lambda i: (0, 0)),
            full, bias, full, full, full,
        ],
        out_specs=[row] * 4,
        out_shape=[o, o, o, o],
        compiler_params=pltpu.CompilerParams(
            dimension_semantics=("parallel",)),
    )(pe, hu1, agg, wu2[NF:], bu.reshape(1, NF), attr2, state2,
      wn[:ATTR], wn[ATTR:ATTR + STATE], wn[ATTR + STATE:], bn.reshape(1, NF),
      we[:NF], we[NF:], wu[:NF])


def _last_body(pe_ref, hu1_ref, agg_ref, wu2, bu, wp, bp, o_ref):
    upd = jnp.maximum(hu1_ref[...] + _dot(agg_ref[...], wu2[...]) + bu[...],
                      0.0)
    pe = pe_ref[...] + upd
    o_ref[...] = _dot(pe, wp[...]) + bp[...]


def _tc_last(pe, hu1, agg, wu2, bu, wp8, bp8):
    row = pl.BlockSpec((NP, NF), lambda i: (i, 0))
    return pl.pallas_call(
        _last_body,
        grid=(2,),
        in_specs=[
            row, row, row,
            pl.BlockSpec((NF, NF), lambda i: (0, 0)),
            pl.BlockSpec((1, NF), lambda i: (0, 0)),
            pl.BlockSpec((NF, 8), lambda i: (0, 0)),
            pl.BlockSpec((1, 8), lambda i: (0, 0)),
        ],
        out_specs=pl.BlockSpec((NP, 8), lambda i: (i, 0)),
        out_shape=jax.ShapeDtypeStruct((BS * NP, 8), jnp.float32),
        compiler_params=pltpu.CompilerParams(
            dimension_semantics=("parallel",)),
    )(pe, hu1, agg, wu2, bu.reshape(1, NF), wp8, bp8.reshape(1, 8))


@functools.lru_cache(maxsize=None)
def _sc_kernels():
    mesh = plsc.VectorSubcoreMesh(core_axis_name="c", subcore_axis_name="s")

    @functools.partial(
        pl.kernel,
        out_type=jax.ShapeDtypeStruct((BS * NP, NF), jnp.float32),
        mesh=mesh,
        scratch_types=[
            pltpu.VMEM((12, 128), jnp.int32),
            pltpu.VMEM((128, NF), jnp.float32),
            pltpu.VMEM((128, NF), jnp.float32),
            pltpu.VMEM((128, NF), jnp.float32),
            pltpu.VMEM((128, NF), jnp.float32),
            pltpu.VMEM((1, NF), jnp.float32),
            pltpu.VMEM_SHARED((NP, NF), jnp.float32),
            pltpu.SemaphoreType.DMA,
            pltpu.SemaphoreType.DMA,
        ],
    )
    def _edge_sc(hw1_hbm, hw2_hbm, idx_hbm, bias_hbm, zeros_hbm, agg_hbm,
                 idx_v, a0_v, a1_v, b0_v, b1_v, bias_v, agg_sh, gsem, ssem):
        c = lax.axis_index("c")
        s = lax.axis_index("s")
        w = c * NS + s
        pltpu.sync_copy(zeros_hbm, agg_sh.at[pl.ds(s * 64, 64)])
        pltpu.sync_copy(idx_hbm.at[w], idx_v)
        pltpu.sync_copy(bias_hbm, bias_v)
        plsc.subcore_barrier()

        bias_regs = [bias_v[0, pl.ds(k * 16, 16)] for k in range(8)]
        ab = ((a0_v, b0_v), (a1_v, b1_v))

        gh = [None] * 4
        sh = [None] * 4

        def gather(j):
            a, b = ab[j % 2]
            return (pltpu.async_copy(hw1_hbm.at[idx_v.at[j]], a, gsem),
                    pltpu.async_copy(hw2_hbm.at[idx_v.at[4 + j]], b, gsem))

        gh[0] = gather(0)
        for j in range(4):
            a_v, b_v = ab[j % 2]
            gh[j][0].wait()
            gh[j][1].wait()
            if j < 3:
                gh[j + 1] = gather(j + 1)
            if j >= 2:
                sh[j - 2].wait()

            @pl.loop(0, 128)
            def _(r):
                for k in range(8):
                    sl = pl.ds(k * 16, 16)
                    v = jnp.maximum(a_v[r, sl] + b_v[r, sl] + bias_regs[k],
                                    0.0)
                    vu = lax.bitcast_convert_type(v, jnp.uint32)
                    vu = ((vu + jnp.uint32(0x7FFF)
                           + ((vu >> jnp.uint32(16)) & jnp.uint32(1)))
                          & jnp.uint32(0xFFFF0000))
                    a_v[r, sl] = lax.bitcast_convert_type(vu, jnp.float32)

            sh[j] = pltpu.async_copy(a_v, agg_sh.at[idx_v.at[8 + j]], ssem,
                                     add=True)
        sh[2].wait()
        sh[3].wait()

        plsc.subcore_barrier()
        pltpu.sync_copy(agg_sh.at[pl.ds(s * 64, 64)],
                        agg_hbm.at[pl.ds(c * NP + s * 64, 64)])

    return _edge_sc


def kernel(attr, state_norm, Rr, Rs, W_node, b_node, W_edge, b_edge,
           W_upd, b_upd, W_pred, b_pred):
    f32, i32 = jnp.float32, jnp.int32

    pad_hot = ((0, 0), (0, 0), (0, NPAD - N))
    idx_all = _extract_indices(jnp.pad(Rr.astype(jnp.int8), pad_hot),
                               jnp.pad(Rs.astype(jnp.int8), pad_hot))

    attr2 = jnp.pad(attr, ((0, 0), (0, NP - N), (0, 0))).reshape(BS * NP, ATTR)
    state2 = jnp.pad(state_norm, ((0, 0), (0, NP - N), (0, 0))
                     ).reshape(BS * NP, STATE)
    zeros64 = jnp.zeros((64, NF), f32)
    w_pred8 = jnp.pad(W_pred, ((0, 0), (0, 8 - OUT)))
    b_pred8 = jnp.pad(b_pred, (0, 8 - OUT))

    edge_sc = _sc_kernels()

    pe = jnp.zeros((BS * NP, NF), f32)
    hw1, hw2, hu1 = _tc_first(attr2, state2, W_node[0], b_node[0],
                              W_edge[0], W_upd[0])
    for l in range(L):
        agg = edge_sc(hw1, hw2, idx_all, b_edge[l].reshape(1, NF), zeros64)
        if l < L - 1:
            pe, hw1, hw2, hu1 = _tc_mid(
                pe, hu1, agg, W_upd[l], b_upd[l], attr2, state2,
                W_node[l + 1], b_node[l + 1], W_edge[l + 1], W_upd[l + 1])
        else:
            out = _tc_last(pe, hu1, agg, W_upd[l][NF:], b_upd[l],
                           w_pred8, b_pred8)

    return out.reshape(BS, NP, 8)[:, :N, :OUT]
